# baseline (speedup 1.0000x reference)
import functools

import jax
import jax.numpy as jnp
from jax import lax
from jax.experimental import pallas as pl
from jax.experimental.pallas import tpu as pltpu
from jax.experimental.pallas import tpu_sc as plsc

DIM = 128
SENT_L = 50
PAD_L = 56
SEG = 2 * PAD_L
BATCH = 4096
ALPHA_COEF = 0.1

_NC = 2
_NS = 16
_NW = _NC * _NS
_B_PER_W = BATCH // _NW

_mesh = plsc.VectorSubcoreMesh(core_axis_name="c", subcore_axis_name="s")


def _gather(embed_hbm, idx, dst, sem):
    return pltpu.make_async_copy(embed_hbm.at[idx], dst, sem)


def _lanesum(v):
    r = v
    for sh in (8, 4, 2, 1):
        idx = lax.iota(jnp.int32, 16) ^ sh
        r = r + r.at[idx].get(mode="promise_in_bounds")
    return r


@functools.partial(
    pl.kernel,
    mesh=_mesh,
    out_type=(
        jax.ShapeDtypeStruct((BATCH, DIM), jnp.float32),
        jax.ShapeDtypeStruct((BATCH, DIM), jnp.float32),
        jax.ShapeDtypeStruct((BATCH, DIM), jnp.float32),
    ),
    scratch_types=[
        pltpu.VMEM((_B_PER_W, SEG), jnp.int32),
        pltpu.VMEM((SEG, DIM), jnp.float32),
        pltpu.VMEM((SEG, DIM), jnp.float32),
        pltpu.VMEM((DIM,), jnp.float32),
        pltpu.VMEM((_B_PER_W, DIM), jnp.float32),
        pltpu.VMEM((_B_PER_W, DIM), jnp.float32),
        pltpu.VMEM((_B_PER_W, DIM), jnp.float32),
        pltpu.SemaphoreType.DMA,
        pltpu.SemaphoreType.DMA,
    ],
)
def _sc_encode(ids_hbm, embed_hbm, h0_hbm, vp_hbm, vh_hbm,
               ids_v, rows0, rows1, e0_v, st_h0, st_vp, st_vh, sem0, sem1):
    wid = lax.axis_index("s") * _NC + lax.axis_index("c")
    base = wid * _B_PER_W

    pltpu.sync_copy(ids_hbm.at[pl.ds(base, _B_PER_W)], ids_v)
    pltpu.sync_copy(embed_hbm.at[0], e0_v)

    e0c = tuple(e0_v[pl.ds(16 * c, 16)] for c in range(8))
    lane = lax.iota(jnp.int32, 16)
    front = jnp.minimum(jnp.maximum(8 - lane, 0), 1)

    zero16 = jnp.zeros((16,), jnp.float32)

    def _process(i, rows, st_slots):
        nz = []
        for k in range(7):
            nz.append(jnp.minimum(ids_v[i, pl.ds(16 * k, 16)], 1))
        bound_nz = nz[3] * front
        nonpad_p = _lanesum(nz[0] + nz[1] + nz[2] + bound_nz)
        nonpad_h = _lanesum(nz[3] - bound_nz + nz[4] + nz[5] + nz[6])
        npp = PAD_L - nonpad_p
        nph = PAD_L - nonpad_h

        def abody(r, carry):
            out = []
            for c in range(8):
                out.append(carry[c] + rows[r, pl.ds(16 * c, 16)])
            for c in range(8):
                out.append(carry[8 + c] + rows[PAD_L + r, pl.ds(16 * c, 16)])
            return tuple(out)

        accs = lax.fori_loop(0, PAD_L, abody, (zero16,) * 16)

        npp_f = npp.astype(jnp.float32)
        nph_f = nph.astype(jnp.float32)
        denp = jnp.maximum((PAD_L - npp).astype(jnp.float32), 1.0)
        denh = jnp.maximum((PAD_L - nph).astype(jnp.float32), 1.0)

        for c in range(8):
            vp_c = (accs[c] - npp_f * e0c[c]) / denp
            vh_c = (accs[8 + c] - nph_f * e0c[c]) / denh
            h0_c = (vh_c - vp_c) + ALPHA_COEF * (
                rows[PAD_L, pl.ds(16 * c, 16)] - rows[0, pl.ds(16 * c, 16)])
            sl = pl.ds(16 * c, 16)
            st_slots[0][i, sl] = h0_c
            st_slots[1][i, sl] = vp_c
            st_slots[2][i, sl] = vh_c

    stages = (st_h0, st_vp, st_vh)

    _gather(embed_hbm, ids_v.at[0], rows0, sem0).start()

    def gbody(g, carry):
        i0 = 2 * g
        i1 = i0 + 1
        _gather(embed_hbm, ids_v.at[i1], rows1, sem1).start()
        _gather(embed_hbm, ids_v.at[i0], rows0, sem0).wait()
        _process(i0, rows0, stages)

        @pl.when(i0 + 2 < _B_PER_W)
        def _():
            _gather(embed_hbm, ids_v.at[i0 + 2], rows0, sem0).start()

        _gather(embed_hbm, ids_v.at[i1], rows1, sem1).wait()
        _process(i1, rows1, stages)
        return carry

    lax.fori_loop(0, _B_PER_W // 2, gbody, 0)

    pltpu.sync_copy(st_h0, h0_hbm.at[pl.ds(base, _B_PER_W)])
    pltpu.sync_copy(st_vp, vp_hbm.at[pl.ds(base, _B_PER_W)])
    pltpu.sync_copy(st_vh, vh_hbm.at[pl.ds(base, _B_PER_W)])


@jax.jit
def kernel(prem_ids, hyp_ids, embed):
    prem = prem_ids.astype(jnp.int32)
    hyp = hyp_ids.astype(jnp.int32)
    pad = ((0, 0), (0, PAD_L - SENT_L))
    ids = jnp.concatenate([jnp.pad(prem, pad), jnp.pad(hyp, pad)], axis=1)
    h0, v_p, v_h = _sc_encode(ids, embed)
    return (h0, v_p, v_h)
